# baseline (speedup 1.0000x reference)
import functools

import jax
import jax.numpy as jnp
from jax import lax
from jax.experimental import pallas as pl
from jax.experimental.pallas import tpu as pltpu
from jax.experimental.pallas import tpu_sc as plsc

NC = 2
NS = 16
NW = NC * NS
K = 80


def _deg_build(n, epw):
  nchunks = epw // K
  main = (n // NS) // 8 * 8
  tail = n - main * NS
  mesh = plsc.VectorSubcoreMesh(core_axis_name="c", subcore_axis_name="s")

  @functools.partial(
      pl.kernel,
      out_type=jax.ShapeDtypeStruct((NC * n,), jnp.float32),
      mesh=mesh,
      scratch_types=[
          pltpu.VMEM((epw // K, K), jnp.int32),
          pltpu.VMEM((K,), jnp.float32),
          pltpu.VMEM((main,), jnp.float32),
          pltpu.VMEM_SHARED((n,), jnp.float32),
          pltpu.SemaphoreType.DMA,
      ],
  )
  def deg_kernel(dst_hbm, ones_hbm, zeros_hbm, out_hbm,
                 didx_all, ones_v, stage, deg_sh, ssem):
    cid = lax.axis_index("c")
    sid = lax.axis_index("s")
    wid = cid * NS + sid

    pltpu.sync_copy(zeros_hbm, stage)
    pltpu.sync_copy(stage, deg_sh.at[pl.ds(sid * main, main)])
    if tail:
      @pl.when(sid == NS - 1)
      def _():
        pltpu.sync_copy(stage.at[pl.ds(0, tail)],
                        deg_sh.at[pl.ds(main * NS, tail)])
    pltpu.sync_copy(ones_hbm, ones_v)
    pltpu.sync_copy(dst_hbm.at[wid], didx_all)
    plsc.subcore_barrier()

    DEPTH = 4

    def start_scat(c):
      pltpu.async_copy(ones_v, deg_sh.at[didx_all.at[c]], ssem, add=True)

    def wait_one():
      pltpu.make_async_copy(ones_v, deg_sh.at[didx_all.at[0]], ssem).wait()

    for c in range(DEPTH):
      start_scat(c)

    def body(c, carry):
      wait_one()
      start_scat(c)
      return carry

    lax.fori_loop(DEPTH, nchunks, body, 0)
    for _ in range(DEPTH):
      wait_one()
    plsc.subcore_barrier()

    pltpu.sync_copy(deg_sh.at[pl.ds(sid * main, main)], stage)
    pltpu.sync_copy(stage, out_hbm.at[pl.ds(cid * n + sid * main, main)])
    if tail:
      @pl.when(sid == NS - 1)
      def _():
        pltpu.sync_copy(deg_sh.at[pl.ds(main * NS, tail)],
                        stage.at[pl.ds(0, tail)])
        pltpu.sync_copy(stage.at[pl.ds(0, tail)],
                        out_hbm.at[pl.ds(cid * n + main * NS, tail)])

  return deg_kernel


CH = 48


def _agg_build(n, epw, d):
  nchunks = epw // K
  main = (n // NS) // 8 * 8
  tail = n - main * NS
  assert main % CH == 0 and tail <= CH
  mesh = plsc.VectorSubcoreMesh(core_axis_name="c", subcore_axis_name="s")

  @functools.partial(
      pl.kernel,
      out_type=jax.ShapeDtypeStruct((NC, n, d), jnp.float32),
      mesh=mesh,
      scratch_types=[
          pltpu.VMEM((epw,), jnp.int32),
          pltpu.VMEM((epw // K, K), jnp.int32),
          pltpu.VMEM((2, K, d), jnp.float32),
          pltpu.VMEM_SHARED((n, d), jnp.float32),
          pltpu.SemaphoreType.DMA,
          pltpu.SemaphoreType.DMA,
          pltpu.SemaphoreType.DMA,
          pltpu.SemaphoreType.DMA,
      ],
  )
  def agg_kernel(src_hbm, dst_hbm, y_hbm, zeros_hbm, out_hbm,
                 sidx_all, didx_all, rows, acc,
                 gsem0, gsem1, ssem0, ssem1):
    cid = lax.axis_index("c")
    sid = lax.axis_index("s")
    wid = cid * NS + sid
    stage = rows.at[0, pl.ds(0, CH)]

    pltpu.sync_copy(zeros_hbm, stage)

    def zbody(j, carry):
      pltpu.sync_copy(stage, acc.at[pl.ds(sid * main + j * CH, CH)])
      return carry

    lax.fori_loop(0, main // CH, zbody, 0)
    if tail:
      @pl.when(sid == NS - 1)
      def _():
        pltpu.sync_copy(rows.at[0, pl.ds(0, tail)],
                        acc.at[pl.ds(main * NS, tail)])

    pltpu.sync_copy(src_hbm.at[pl.ds(wid * epw, epw)], sidx_all)
    pltpu.sync_copy(dst_hbm.at[wid], didx_all)
    plsc.subcore_barrier()

    r0, r1 = rows.at[0], rows.at[1]

    def start_gather(c, rb, sem):
      pltpu.async_copy(y_hbm.at[sidx_all.at[pl.ds(c * K, K)]], rb, sem)

    def wait_gather(rb, sem):
      pltpu.make_async_copy(y_hbm.at[sidx_all.at[pl.ds(0, K)]], rb, sem).wait()

    def start_scat(c, rb, sem):
      pltpu.async_copy(rb, acc.at[didx_all.at[c]], sem, add=True)

    def wait_scat(rb, sem):
      pltpu.make_async_copy(rb, acc.at[didx_all.at[0]], sem).wait()

    start_gather(0, r0, gsem0)
    wait_gather(r0, gsem0)
    start_scat(0, r0, ssem0)
    start_gather(1, r1, gsem1)
    wait_gather(r1, gsem1)
    start_scat(1, r1, ssem1)
    wait_scat(r0, ssem0)
    start_gather(2, r0, gsem0)

    def body(i, carry):
      a = 2 * i
      wait_gather(r0, gsem0)
      start_scat(a, r0, ssem0)
      wait_scat(r1, ssem1)
      start_gather(a + 1, r1, gsem1)
      wait_gather(r1, gsem1)
      start_scat(a + 1, r1, ssem1)
      wait_scat(r0, ssem0)
      start_gather(a + 2, r0, gsem0)
      return carry

    lax.fori_loop(1, (nchunks - 1) // 2, body, 0)
    wait_gather(r0, gsem0)
    start_scat(nchunks - 1, r0, ssem0)
    wait_scat(r1, ssem1)
    wait_scat(r0, ssem0)
    plsc.subcore_barrier()

    def wbody(j, carry):
      off = sid * main + j * CH
      pltpu.sync_copy(acc.at[pl.ds(off, CH)], stage)
      pltpu.sync_copy(stage, out_hbm.at[cid, pl.ds(off, CH)])
      return carry

    lax.fori_loop(0, main // CH, wbody, 0)
    if tail:
      @pl.when(sid == NS - 1)
      def _():
        pltpu.sync_copy(acc.at[pl.ds(main * NS, tail)],
                        rows.at[0, pl.ds(0, tail)])
        pltpu.sync_copy(rows.at[0, pl.ds(0, tail)],
                        out_hbm.at[cid, pl.ds(main * NS, tail)])

  return agg_kernel


def _linear(x, wt, degp_t):
  n, din = x.shape
  dout = wt.shape[1]
  rows = 1000

  def body(x_ref, wt_ref, dp_ref, y_ref):
    deg = dp_ref[:, 0:1] + dp_ref[:, 1:2]
    dis = jnp.where(deg > 0, lax.rsqrt(jnp.where(deg > 0, deg, 1.0)), 0.0)
    y_ref[...] = jnp.dot(x_ref[...] * dis, wt_ref[...],
                         preferred_element_type=jnp.float32)

  return pl.pallas_call(
      body,
      grid=(n // rows,),
      in_specs=[
          pl.BlockSpec((rows, din), lambda i: (i, 0)),
          pl.BlockSpec((din, dout), lambda i: (0, 0)),
          pl.BlockSpec((rows, NC), lambda i: (i, 0)),
      ],
      out_specs=pl.BlockSpec((rows, dout), lambda i: (i, 0)),
      out_shape=jax.ShapeDtypeStruct((n, dout), jnp.float32),
  )(x, wt, degp_t)


def _finalize(p0, p1, degp_t, b2):
  n, dout = p0.shape
  rows = 1000

  def body(p0_ref, p1_ref, dp_ref, b_ref, o_ref):
    deg = dp_ref[:, 0:1] + dp_ref[:, 1:2]
    dis = jnp.where(deg > 0, lax.rsqrt(jnp.where(deg > 0, deg, 1.0)), 0.0)
    o_ref[...] = (p0_ref[...] + p1_ref[...]) * dis + b_ref[...]

  return pl.pallas_call(
      body,
      grid=(n // rows,),
      in_specs=[
          pl.BlockSpec((rows, dout), lambda i: (i, 0)),
          pl.BlockSpec((rows, dout), lambda i: (i, 0)),
          pl.BlockSpec((rows, NC), lambda i: (i, 0)),
          pl.BlockSpec((1, dout), lambda i: (0, 0)),
      ],
      out_specs=pl.BlockSpec((rows, dout), lambda i: (i, 0)),
      out_shape=jax.ShapeDtypeStruct((n, dout), jnp.float32),
  )(p0, p1, degp_t, b2)


def kernel(input_feature, edge_index, W, b):
  x = input_feature
  n, _ = x.shape
  dout = W.shape[0]
  e = edge_index.shape[1]
  assert e % (NW * K) == 0 and n % NS == 0

  epw = e // NW
  nchunks = epw // K
  assert nchunks % 2 == 1 and nchunks >= 5
  src1 = edge_index[0]
  dst3 = jnp.reshape(edge_index[1], (NW, nchunks, K))
  main = (n // NS) // 8 * 8

  ones = jnp.ones((K,), jnp.float32)
  zeros1 = jnp.zeros((main,), jnp.float32)
  zeros2 = jnp.zeros((CH, dout), jnp.float32)

  degp = _deg_build(n, epw)(dst3, ones, zeros1)
  degp_t = jnp.reshape(degp, (NC, n)).T
  y = _linear(x, W.T, degp_t)
  p = _agg_build(n, epw, dout)(src1, dst3, y, zeros2)
  return _finalize(p[0], p[1], degp_t, jnp.reshape(b, (1, dout)))


# speedup vs baseline: 1.0926x; 1.0926x over previous
import functools

import jax
import jax.numpy as jnp
from jax import lax
from jax.experimental import pallas as pl
from jax.experimental.pallas import tpu as pltpu
from jax.experimental.pallas import tpu_sc as plsc

NC = 2
NS = 16
NW = NC * NS
K = 128
NX = 32
CH = 48


def _deg_build(n, n_acc, epw):
  nchunks = epw // K
  main = (n_acc // NS) // 8 * 8
  tail = n_acc - main * NS
  mainw = (n // NS) // 8 * 8
  tailw = n - mainw * NS
  assert nchunks >= 5 and tail >= 0 and tailw >= 0
  mesh = plsc.VectorSubcoreMesh(core_axis_name="c", subcore_axis_name="s")

  @functools.partial(
      pl.kernel,
      out_type=jax.ShapeDtypeStruct((NC * n,), jnp.float32),
      mesh=mesh,
      scratch_types=[
          pltpu.VMEM((epw // K, K), jnp.int32),
          pltpu.VMEM((K,), jnp.float32),
          pltpu.VMEM((main,), jnp.float32),
          pltpu.VMEM_SHARED((n_acc,), jnp.float32),
          pltpu.SemaphoreType.DMA,
      ],
  )
  def deg_kernel(dst_hbm, ones_hbm, zeros_hbm, out_hbm,
                 didx_all, ones_v, stage, deg_sh, ssem):
    cid = lax.axis_index("c")
    sid = lax.axis_index("s")
    wid = cid * NS + sid

    pltpu.sync_copy(zeros_hbm, stage)
    pltpu.sync_copy(stage, deg_sh.at[pl.ds(sid * main, main)])
    if tail:
      @pl.when(sid == NS - 1)
      def _():
        pltpu.sync_copy(stage.at[pl.ds(0, tail)],
                        deg_sh.at[pl.ds(main * NS, tail)])
    pltpu.sync_copy(ones_hbm, ones_v)
    pltpu.sync_copy(dst_hbm.at[wid], didx_all)
    plsc.subcore_barrier()

    DEPTH = 4

    def start_scat(c):
      pltpu.async_copy(ones_v, deg_sh.at[didx_all.at[c]], ssem, add=True)

    def wait_one():
      pltpu.make_async_copy(ones_v, deg_sh.at[didx_all.at[0]], ssem).wait()

    for c in range(DEPTH):
      start_scat(c)

    def body(c, carry):
      wait_one()
      start_scat(c)
      return carry

    lax.fori_loop(DEPTH, nchunks, body, 0)
    for _ in range(DEPTH):
      wait_one()
    plsc.subcore_barrier()

    pltpu.sync_copy(deg_sh.at[pl.ds(sid * mainw, mainw)], stage.at[pl.ds(0, mainw)])
    pltpu.sync_copy(stage.at[pl.ds(0, mainw)],
                    out_hbm.at[pl.ds(cid * n + sid * mainw, mainw)])
    if tailw:
      @pl.when(sid == NS - 1)
      def _():
        pltpu.sync_copy(deg_sh.at[pl.ds(mainw * NS, tailw)],
                        stage.at[pl.ds(0, tailw)])
        pltpu.sync_copy(stage.at[pl.ds(0, tailw)],
                        out_hbm.at[pl.ds(cid * n + mainw * NS, tailw)])

  return deg_kernel


def _agg_build(n, n_acc, epw, d):
  nchunks = epw // K
  main = (n_acc // NS) // 8 * 8
  tail = n_acc - main * NS
  mainw = (n // NS) // 8 * 8
  tailw = n - mainw * NS
  assert nchunks % 2 == 1 and nchunks >= 5
  assert mainw % CH == 0 and tailw <= CH and tail <= CH
  phase = (nchunks // 2 + 1) // 2 * 2
  half = max(phase, nchunks - phase) * K
  reload_i = (phase - 2) // 2
  mesh = plsc.VectorSubcoreMesh(core_axis_name="c", subcore_axis_name="s")

  @functools.partial(
      pl.kernel,
      out_type=jax.ShapeDtypeStruct((NC, n, d), jnp.float32),
      mesh=mesh,
      scratch_types=[
          pltpu.VMEM((half,), jnp.int32),
          pltpu.VMEM((epw // K, K), jnp.int32),
          pltpu.VMEM((2, K, d), jnp.float32),
          pltpu.VMEM_SHARED((n_acc, d), jnp.float32),
          pltpu.SemaphoreType.DMA,
          pltpu.SemaphoreType.DMA,
          pltpu.SemaphoreType.DMA,
          pltpu.SemaphoreType.DMA,
      ],
  )
  def agg_kernel(src_hbm, dst_hbm, y_hbm, zeros_hbm, out_hbm,
                 sidx_half, didx_all, rows, acc,
                 gsem0, gsem1, ssem0, ssem1):
    cid = lax.axis_index("c")
    sid = lax.axis_index("s")
    wid = cid * NS + sid
    stage = rows.at[0, pl.ds(0, CH)]

    pltpu.sync_copy(zeros_hbm, stage)

    def zbody(j, carry):
      pltpu.sync_copy(stage, acc.at[pl.ds(sid * main + j * CH, CH)])
      return carry

    lax.fori_loop(0, main // CH, zbody, 0)
    if tail:
      @pl.when(sid == NS - 1)
      def _():
        pltpu.sync_copy(rows.at[0, pl.ds(0, tail)],
                        acc.at[pl.ds(main * NS, tail)])

    pltpu.sync_copy(src_hbm.at[pl.ds(wid * epw, phase * K)],
                    sidx_half.at[pl.ds(0, phase * K)])
    pltpu.sync_copy(dst_hbm.at[wid], didx_all)
    plsc.subcore_barrier()

    r0, r1 = rows.at[0], rows.at[1]

    def start_gather(c, rb, sem):
      off = jnp.where(c < phase, c * K, (c - phase) * K)
      pltpu.async_copy(y_hbm.at[sidx_half.at[pl.ds(off, K)]], rb, sem)

    def wait_gather(rb, sem):
      pltpu.make_async_copy(y_hbm.at[sidx_half.at[pl.ds(0, K)]], rb, sem).wait()

    def start_scat(c, rb, sem):
      pltpu.async_copy(rb, acc.at[didx_all.at[c]], sem, add=True)

    def wait_scat(rb, sem):
      pltpu.make_async_copy(rb, acc.at[didx_all.at[0]], sem).wait()

    start_gather(0, r0, gsem0)
    wait_gather(r0, gsem0)
    start_scat(0, r0, ssem0)
    start_gather(1, r1, gsem1)
    wait_gather(r1, gsem1)
    start_scat(1, r1, ssem1)
    wait_scat(r0, ssem0)
    start_gather(2, r0, gsem0)

    def body(i, carry):
      a = 2 * i
      wait_scat(r1, ssem1)
      start_gather(a + 1, r1, gsem1)
      wait_gather(r0, gsem0)
      start_scat(a, r0, ssem0)
      wait_gather(r1, gsem1)

      @pl.when(i == reload_i)
      def _():
        pltpu.sync_copy(
            src_hbm.at[pl.ds(wid * epw + phase * K, (nchunks - phase) * K)],
            sidx_half.at[pl.ds(0, (nchunks - phase) * K)])

      start_scat(a + 1, r1, ssem1)
      wait_scat(r0, ssem0)
      start_gather(a + 2, r0, gsem0)
      return carry

    lax.fori_loop(1, (nchunks - 1) // 2, body, 0)
    wait_gather(r0, gsem0)
    start_scat(nchunks - 1, r0, ssem0)
    wait_scat(r1, ssem1)
    wait_scat(r0, ssem0)
    plsc.subcore_barrier()

    def wbody(j, carry):
      off = sid * mainw + j * CH
      pltpu.sync_copy(acc.at[pl.ds(off, CH)], stage)
      pltpu.sync_copy(stage, out_hbm.at[cid, pl.ds(off, CH)])
      return carry

    lax.fori_loop(0, mainw // CH, wbody, 0)
    if tailw:
      @pl.when(sid == NS - 1)
      def _():
        pltpu.sync_copy(acc.at[pl.ds(mainw * NS, tailw)],
                        rows.at[0, pl.ds(0, tailw)])
        pltpu.sync_copy(rows.at[0, pl.ds(0, tailw)],
                        out_hbm.at[cid, pl.ds(mainw * NS, tailw)])

  return agg_kernel


def _linear(x, wt, degp_t):
  n, din = x.shape
  dout = wt.shape[1]
  rows = 1000

  def body(x_ref, wt_ref, dp_ref, y_ref):
    deg = dp_ref[:, 0:1] + dp_ref[:, 1:2]
    dis = jnp.where(deg > 0, lax.rsqrt(jnp.where(deg > 0, deg, 1.0)), 0.0)
    y_ref[...] = jnp.dot(x_ref[...] * dis, wt_ref[...],
                         preferred_element_type=jnp.float32)

  return pl.pallas_call(
      body,
      grid=(n // rows,),
      in_specs=[
          pl.BlockSpec((rows, din), lambda i: (i, 0)),
          pl.BlockSpec((din, dout), lambda i: (0, 0)),
          pl.BlockSpec((rows, NC), lambda i: (i, 0)),
      ],
      out_specs=pl.BlockSpec((rows, dout), lambda i: (i, 0)),
      out_shape=jax.ShapeDtypeStruct((n, dout), jnp.float32),
  )(x, wt, degp_t)


def _finalize(p0, p1, degp_t, b2):
  n, dout = p0.shape
  rows = 1000

  def body(p0_ref, p1_ref, dp_ref, b_ref, o_ref):
    deg = dp_ref[:, 0:1] + dp_ref[:, 1:2]
    dis = jnp.where(deg > 0, lax.rsqrt(jnp.where(deg > 0, deg, 1.0)), 0.0)
    o_ref[...] = (p0_ref[...] + p1_ref[...]) * dis + b_ref[...]

  return pl.pallas_call(
      body,
      grid=(n // rows,),
      in_specs=[
          pl.BlockSpec((rows, dout), lambda i: (i, 0)),
          pl.BlockSpec((rows, dout), lambda i: (i, 0)),
          pl.BlockSpec((rows, NC), lambda i: (i, 0)),
          pl.BlockSpec((1, dout), lambda i: (0, 0)),
      ],
      out_specs=pl.BlockSpec((rows, dout), lambda i: (i, 0)),
      out_shape=jax.ShapeDtypeStruct((n, dout), jnp.float32),
  )(p0, p1, degp_t, b2)


def kernel(input_feature, edge_index, W, b):
  x = input_feature
  n, _ = x.shape
  dout = W.shape[0]
  e = edge_index.shape[1]
  assert n % NS == 0 and (n + NX) % 8 == 0

  epw0 = -(-e // NW)
  nchunks = -(-epw0 // K)
  if nchunks % 2 == 0:
    nchunks += 1
  epw = nchunks * K
  pad = NW * epw - e
  n_acc = n + NX
  if pad:
    ar = jnp.arange(pad, dtype=jnp.int32)
    src1 = jnp.concatenate([edge_index[0], (ar * 7919) % n])
    dstp = jnp.concatenate([edge_index[1], n + (ar % NX)])
  else:
    src1 = edge_index[0]
    dstp = edge_index[1]
  dst3 = jnp.reshape(dstp, (NW, nchunks, K))
  main = ((n + NX) // NS) // 8 * 8

  ones = jnp.ones((K,), jnp.float32)
  zeros1 = jnp.zeros((main,), jnp.float32)
  zeros2 = jnp.zeros((CH, dout), jnp.float32)

  degp = _deg_build(n, n_acc, epw)(dst3, ones, zeros1)
  degp_t = jnp.reshape(degp, (NC, n)).T
  y = _linear(x, W.T, degp_t)
  p = _agg_build(n, n_acc, epw, dout)(src1, dst3, y, zeros2)
  return _finalize(p[0], p[1], degp_t, jnp.reshape(b, (1, dout)))


# speedup vs baseline: 1.0968x; 1.0039x over previous
import functools

import jax
import jax.numpy as jnp
from jax import lax
from jax.experimental import pallas as pl
from jax.experimental.pallas import tpu as pltpu
from jax.experimental.pallas import tpu_sc as plsc

NC = 2
NS = 16
NW = NC * NS
K = 128
NX = 32
CH = 48


def _deg_build(n, n_acc, epw):
  nchunks = epw // K
  main = (n_acc // NS) // 8 * 8
  tail = n_acc - main * NS
  mainw = (n // NS) // 8 * 8
  tailw = n - mainw * NS
  assert nchunks >= 5 and tail >= 0 and tailw >= 0
  mesh = plsc.VectorSubcoreMesh(core_axis_name="c", subcore_axis_name="s")

  @functools.partial(
      pl.kernel,
      out_type=jax.ShapeDtypeStruct((NC * n,), jnp.float32),
      mesh=mesh,
      scratch_types=[
          pltpu.VMEM((epw // K, K), jnp.int32),
          pltpu.VMEM((K,), jnp.float32),
          pltpu.VMEM((main,), jnp.float32),
          pltpu.VMEM_SHARED((n_acc,), jnp.float32),
          pltpu.SemaphoreType.DMA,
      ],
  )
  def deg_kernel(dst_hbm, ones_hbm, zeros_hbm, out_hbm,
                 didx_all, ones_v, stage, deg_sh, ssem):
    cid = lax.axis_index("c")
    sid = lax.axis_index("s")
    wid = cid * NS + sid

    pltpu.sync_copy(zeros_hbm, stage)
    pltpu.sync_copy(stage, deg_sh.at[pl.ds(sid * main, main)])
    if tail:
      @pl.when(sid == NS - 1)
      def _():
        pltpu.sync_copy(stage.at[pl.ds(0, tail)],
                        deg_sh.at[pl.ds(main * NS, tail)])
    pltpu.sync_copy(ones_hbm, ones_v)
    pltpu.sync_copy(dst_hbm.at[wid], didx_all)
    plsc.subcore_barrier()

    DEPTH = 8

    def start_scat(c):
      pltpu.async_copy(ones_v, deg_sh.at[didx_all.at[c]], ssem, add=True)

    def wait_one():
      pltpu.make_async_copy(ones_v, deg_sh.at[didx_all.at[0]], ssem).wait()

    for c in range(DEPTH):
      start_scat(c)

    def body(c, carry):
      wait_one()
      start_scat(c)
      return carry

    lax.fori_loop(DEPTH, nchunks, body, 0)
    for _ in range(DEPTH):
      wait_one()
    plsc.subcore_barrier()

    pltpu.sync_copy(deg_sh.at[pl.ds(sid * mainw, mainw)], stage.at[pl.ds(0, mainw)])
    pltpu.sync_copy(stage.at[pl.ds(0, mainw)],
                    out_hbm.at[pl.ds(cid * n + sid * mainw, mainw)])
    if tailw:
      @pl.when(sid == NS - 1)
      def _():
        pltpu.sync_copy(deg_sh.at[pl.ds(mainw * NS, tailw)],
                        stage.at[pl.ds(0, tailw)])
        pltpu.sync_copy(stage.at[pl.ds(0, tailw)],
                        out_hbm.at[pl.ds(cid * n + mainw * NS, tailw)])

  return deg_kernel


def _agg_build(n, n_acc, epw, d):
  nchunks = epw // K
  main = (n_acc // NS) // 8 * 8
  tail = n_acc - main * NS
  mainw = (n // NS) // 8 * 8
  tailw = n - mainw * NS
  assert nchunks % 2 == 1 and nchunks >= 5
  assert mainw % CH == 0 and tailw <= CH and tail <= CH
  phase = (nchunks // 2 + 1) // 2 * 2
  half = max(phase, nchunks - phase) * K
  reload_i = (phase - 2) // 2
  mesh = plsc.VectorSubcoreMesh(core_axis_name="c", subcore_axis_name="s")

  @functools.partial(
      pl.kernel,
      out_type=jax.ShapeDtypeStruct((NC, n, d), jnp.float32),
      mesh=mesh,
      scratch_types=[
          pltpu.VMEM((half,), jnp.int32),
          pltpu.VMEM((epw // K, K), jnp.int32),
          pltpu.VMEM((2, K, d), jnp.float32),
          pltpu.VMEM_SHARED((n_acc, d), jnp.float32),
          pltpu.SemaphoreType.DMA,
          pltpu.SemaphoreType.DMA,
          pltpu.SemaphoreType.DMA,
          pltpu.SemaphoreType.DMA,
      ],
  )
  def agg_kernel(src_hbm, dst_hbm, y_hbm, zeros_hbm, out_hbm,
                 sidx_half, didx_all, rows, acc,
                 gsem0, gsem1, ssem0, ssem1):
    cid = lax.axis_index("c")
    sid = lax.axis_index("s")
    wid = cid * NS + sid
    stage = rows.at[0, pl.ds(0, CH)]

    pltpu.sync_copy(zeros_hbm, stage)

    def zbody(j, carry):
      pltpu.sync_copy(stage, acc.at[pl.ds(sid * main + j * CH, CH)])
      return carry

    lax.fori_loop(0, main // CH, zbody, 0)
    if tail:
      @pl.when(sid == NS - 1)
      def _():
        pltpu.sync_copy(rows.at[0, pl.ds(0, tail)],
                        acc.at[pl.ds(main * NS, tail)])

    pltpu.sync_copy(src_hbm.at[pl.ds(wid * epw, phase * K)],
                    sidx_half.at[pl.ds(0, phase * K)])
    pltpu.sync_copy(dst_hbm.at[wid], didx_all)
    plsc.subcore_barrier()

    r0, r1 = rows.at[0], rows.at[1]

    def start_gather(c, rb, sem):
      off = jnp.where(c < phase, c * K, (c - phase) * K)
      pltpu.async_copy(y_hbm.at[sidx_half.at[pl.ds(off, K)]], rb, sem)

    def wait_gather(rb, sem):
      pltpu.make_async_copy(y_hbm.at[sidx_half.at[pl.ds(0, K)]], rb, sem).wait()

    def start_scat(c, rb, sem):
      pltpu.async_copy(rb, acc.at[didx_all.at[c]], sem, add=True)

    def wait_scat(rb, sem):
      pltpu.make_async_copy(rb, acc.at[didx_all.at[0]], sem).wait()

    start_gather(0, r0, gsem0)
    wait_gather(r0, gsem0)
    start_scat(0, r0, ssem0)
    start_gather(1, r1, gsem1)
    wait_gather(r1, gsem1)
    start_scat(1, r1, ssem1)
    wait_scat(r0, ssem0)
    start_gather(2, r0, gsem0)

    def body(i, carry):
      a = 2 * i
      wait_scat(r1, ssem1)
      start_gather(a + 1, r1, gsem1)
      wait_gather(r0, gsem0)
      start_scat(a, r0, ssem0)
      wait_gather(r1, gsem1)

      @pl.when(i == reload_i)
      def _():
        pltpu.sync_copy(
            src_hbm.at[pl.ds(wid * epw + phase * K, (nchunks - phase) * K)],
            sidx_half.at[pl.ds(0, (nchunks - phase) * K)])

      start_scat(a + 1, r1, ssem1)
      wait_scat(r0, ssem0)
      start_gather(a + 2, r0, gsem0)
      return carry

    lax.fori_loop(1, (nchunks - 1) // 2, body, 0)
    wait_gather(r0, gsem0)
    start_scat(nchunks - 1, r0, ssem0)
    wait_scat(r1, ssem1)
    wait_scat(r0, ssem0)
    plsc.subcore_barrier()

    def wbody(j, carry):
      off = sid * mainw + j * CH
      pltpu.sync_copy(acc.at[pl.ds(off, CH)], stage)
      pltpu.sync_copy(stage, out_hbm.at[cid, pl.ds(off, CH)])
      return carry

    lax.fori_loop(0, mainw // CH, wbody, 0)
    if tailw:
      @pl.when(sid == NS - 1)
      def _():
        pltpu.sync_copy(acc.at[pl.ds(mainw * NS, tailw)],
                        rows.at[0, pl.ds(0, tailw)])
        pltpu.sync_copy(rows.at[0, pl.ds(0, tailw)],
                        out_hbm.at[cid, pl.ds(mainw * NS, tailw)])

  return agg_kernel


def _linear(x, wt, degp_t):
  n, din = x.shape
  dout = wt.shape[1]
  rows = 1000

  def body(x_ref, wt_ref, dp_ref, y_ref):
    deg = dp_ref[:, 0:1] + dp_ref[:, 1:2]
    dis = jnp.where(deg > 0, lax.rsqrt(jnp.where(deg > 0, deg, 1.0)), 0.0)
    y_ref[...] = jnp.dot(x_ref[...] * dis, wt_ref[...],
                         preferred_element_type=jnp.float32)

  return pl.pallas_call(
      body,
      grid=(n // rows,),
      in_specs=[
          pl.BlockSpec((rows, din), lambda i: (i, 0)),
          pl.BlockSpec((din, dout), lambda i: (0, 0)),
          pl.BlockSpec((rows, NC), lambda i: (i, 0)),
      ],
      out_specs=pl.BlockSpec((rows, dout), lambda i: (i, 0)),
      out_shape=jax.ShapeDtypeStruct((n, dout), jnp.float32),
  )(x, wt, degp_t)


def _finalize(p0, p1, degp_t, b2):
  n, dout = p0.shape
  rows = 1000

  def body(p0_ref, p1_ref, dp_ref, b_ref, o_ref):
    deg = dp_ref[:, 0:1] + dp_ref[:, 1:2]
    dis = jnp.where(deg > 0, lax.rsqrt(jnp.where(deg > 0, deg, 1.0)), 0.0)
    o_ref[...] = (p0_ref[...] + p1_ref[...]) * dis + b_ref[...]

  return pl.pallas_call(
      body,
      grid=(n // rows,),
      in_specs=[
          pl.BlockSpec((rows, dout), lambda i: (i, 0)),
          pl.BlockSpec((rows, dout), lambda i: (i, 0)),
          pl.BlockSpec((rows, NC), lambda i: (i, 0)),
          pl.BlockSpec((1, dout), lambda i: (0, 0)),
      ],
      out_specs=pl.BlockSpec((rows, dout), lambda i: (i, 0)),
      out_shape=jax.ShapeDtypeStruct((n, dout), jnp.float32),
  )(p0, p1, degp_t, b2)


def kernel(input_feature, edge_index, W, b):
  x = input_feature
  n, _ = x.shape
  dout = W.shape[0]
  e = edge_index.shape[1]
  assert n % NS == 0 and (n + NX) % 8 == 0

  epw0 = -(-e // NW)
  nchunks = -(-epw0 // K)
  if nchunks % 2 == 0:
    nchunks += 1
  epw = nchunks * K
  pad = NW * epw - e
  n_acc = n + NX
  if pad:
    ar = jnp.arange(pad, dtype=jnp.int32)
    src1 = jnp.concatenate([edge_index[0], (ar * 7919) % n])
    dstp = jnp.concatenate([edge_index[1], n + (ar % NX)])
  else:
    src1 = edge_index[0]
    dstp = edge_index[1]
  dst3 = jnp.reshape(dstp, (NW, nchunks, K))
  main = ((n + NX) // NS) // 8 * 8

  ones = jnp.ones((K,), jnp.float32)
  zeros1 = jnp.zeros((main,), jnp.float32)
  zeros2 = jnp.zeros((CH, dout), jnp.float32)

  degp = _deg_build(n, n_acc, epw)(dst3, ones, zeros1)
  degp_t = jnp.reshape(degp, (NC, n)).T
  y = _linear(x, W.T, degp_t)
  p = _agg_build(n, n_acc, epw, dout)(src1, dst3, y, zeros2)
  return _finalize(p[0], p[1], degp_t, jnp.reshape(b, (1, dout)))
